# baseline (speedup 1.0000x reference)
import jax
import jax.numpy as jnp
from jax.experimental import pallas as pl
from jax.experimental.pallas import tpu as pltpu

U = 10000
I = 10000
G = 1000
D = 128
N = U + I + G

MB = 200
NM = G // MB
CB = 5120
NC = (N + CB - 1) // CB


def _msg_kernel(uh_ref, ih_ref, ue_ref, ie_ref, wt_ref, b_ref, msg_ref):
    bf = jnp.bfloat16
    u_msg = jnp.dot(uh_ref[...].astype(bf), ue_ref[...].astype(bf),
                    preferred_element_type=jnp.float32)
    i_msg = jnp.dot(ih_ref[...].astype(bf), ie_ref[...].astype(bf),
                    preferred_element_type=jnp.float32)
    msg_ref[...] = (
        jnp.dot(u_msg, wt_ref[:D, :], preferred_element_type=jnp.float32)
        + jnp.dot(i_msg, wt_ref[D:, :], preferred_element_type=jnp.float32)
        + b_ref[...]
    )


def _norm_kernel(fht_ref, msg_ref, out_ref):
    bf = jnp.bfloat16
    out_ref[...] = jax.lax.dot_general(
        fht_ref[...].astype(bf), msg_ref[...].astype(bf),
        (((0,), (0,)), ((), ())),
        preferred_element_type=jnp.float32)


def kernel(user_emb, item_emb, group_emb, user_hyper_graph,
           item_hyper_graph, full_hyper, W_agg, b_agg):
    wt = W_agg.T
    b2 = b_agg.reshape(1, D)
    fh_t = full_hyper.T

    msg = pl.pallas_call(
        _msg_kernel,
        grid=(NM,),
        in_specs=[
            pl.BlockSpec((MB, U), lambda m: (m, 0)),
            pl.BlockSpec((MB, I), lambda m: (m, 0)),
            pl.BlockSpec((U, D), lambda m: (0, 0)),
            pl.BlockSpec((I, D), lambda m: (0, 0)),
            pl.BlockSpec((2 * D, D), lambda m: (0, 0)),
            pl.BlockSpec((1, D), lambda m: (0, 0)),
        ],
        out_specs=pl.BlockSpec((MB, D), lambda m: (m, 0)),
        out_shape=jax.ShapeDtypeStruct((G, D), jnp.float32),
        compiler_params=pltpu.CompilerParams(
            dimension_semantics=("arbitrary",)),
    )(user_hyper_graph, item_hyper_graph, user_emb, item_emb, wt, b2)

    norm_t = pl.pallas_call(
        _norm_kernel,
        grid=(NC,),
        in_specs=[
            pl.BlockSpec((G, CB), lambda c: (0, c)),
            pl.BlockSpec((G, D), lambda c: (0, 0)),
        ],
        out_specs=pl.BlockSpec((CB, D), lambda c: (c, 0)),
        out_shape=jax.ShapeDtypeStruct((N, D), jnp.float32),
        compiler_params=pltpu.CompilerParams(
            dimension_semantics=("arbitrary",)),
    )(fh_t, msg)

    return (norm_t, msg)


# speedup vs baseline: 1.0422x; 1.0422x over previous
import jax
import jax.numpy as jnp
from jax.experimental import pallas as pl
from jax.experimental.pallas import tpu as pltpu

U = 10000
I = 10000
G = 1000
D = 128
N = U + I + G

MB = 200
NM = G // MB
HB = 1792
CB = 2 * HB
NC = (N + CB - 1) // CB


def _msg_kernel(uh_ref, ih_ref, ue_ref, ie_ref, wt_ref, b_ref, msg_ref):
    bf = jnp.bfloat16
    u_msg = jnp.dot(uh_ref[...].astype(bf), ue_ref[...].astype(bf),
                    preferred_element_type=jnp.float32)
    i_msg = jnp.dot(ih_ref[...].astype(bf), ie_ref[...].astype(bf),
                    preferred_element_type=jnp.float32)
    msg_ref[...] = (
        jnp.dot(u_msg, wt_ref[:D, :], preferred_element_type=jnp.float32)
        + jnp.dot(i_msg, wt_ref[D:, :], preferred_element_type=jnp.float32)
        + b_ref[...]
    )


def _norm_kernel(fht0_ref, fht1_ref, msg_ref, out_ref):
    bf = jnp.bfloat16
    m = msg_ref[...].astype(bf)
    out_ref[:HB, :] = jax.lax.dot_general(
        fht0_ref[...].astype(bf), m, (((0,), (0,)), ((), ())),
        preferred_element_type=jnp.float32)
    out_ref[HB:, :] = jax.lax.dot_general(
        fht1_ref[...].astype(bf), m, (((0,), (0,)), ((), ())),
        preferred_element_type=jnp.float32)


def kernel(user_emb, item_emb, group_emb, user_hyper_graph,
           item_hyper_graph, full_hyper, W_agg, b_agg):
    wt = W_agg.T
    b2 = b_agg.reshape(1, D)
    fh_t = full_hyper.T

    msg = pl.pallas_call(
        _msg_kernel,
        grid=(NM,),
        in_specs=[
            pl.BlockSpec((MB, U), lambda m: (m, 0)),
            pl.BlockSpec((MB, I), lambda m: (m, 0)),
            pl.BlockSpec((U, D), lambda m: (0, 0)),
            pl.BlockSpec((I, D), lambda m: (0, 0)),
            pl.BlockSpec((2 * D, D), lambda m: (0, 0)),
            pl.BlockSpec((1, D), lambda m: (0, 0)),
        ],
        out_specs=pl.BlockSpec((MB, D), lambda m: (m, 0)),
        out_shape=jax.ShapeDtypeStruct((G, D), jnp.float32),
        compiler_params=pltpu.CompilerParams(
            dimension_semantics=("arbitrary",)),
    )(user_hyper_graph, item_hyper_graph, user_emb, item_emb, wt, b2)

    norm_t = pl.pallas_call(
        _norm_kernel,
        grid=(NC,),
        in_specs=[
            pl.BlockSpec((G, HB), lambda c: (0, 2 * c)),
            pl.BlockSpec((G, HB), lambda c: (0, 2 * c + 1)),
            pl.BlockSpec((G, D), lambda c: (0, 0)),
        ],
        out_specs=pl.BlockSpec((CB, D), lambda c: (c, 0)),
        out_shape=jax.ShapeDtypeStruct((N, D), jnp.float32),
        compiler_params=pltpu.CompilerParams(
            dimension_semantics=("arbitrary",)),
    )(fh_t, fh_t, msg)

    return (norm_t, msg)
